# baseline (speedup 1.0000x reference)
import jax
import jax.numpy as jnp
from jax.experimental import pallas as pl
from jax.experimental.pallas import tpu as pltpu

D_MODEL = 768
D_FF = 2048
N_EXPERTS = 8
TOP_K = 2
AUX_COEF = 0.01
N_TOKENS = 2048
TM = 128
L_SLOTS = N_TOKENS * TOP_K + N_EXPERTS * TM
N_TILES = L_SLOTS // TM


def _router_body(x_ref, rw_ref, aux_ref, s0_ref, s1_ref, w1_ref, w2_ref,
                 ef_ref, ff_ref):
    x = x_ref[...]
    rw = rw_ref[...]
    dn = (((1,), (1,)), ((), ()))
    logits = jax.lax.dot_general(x, rw, dn, preferred_element_type=jnp.float32)
    p = jax.nn.softmax(logits, axis=-1)
    n, e = p.shape
    lane = jax.lax.broadcasted_iota(jnp.int32, (n, e), 1)
    m1 = jnp.max(p, axis=1, keepdims=True)
    i1 = jnp.min(jnp.where(p >= m1, lane, e), axis=1, keepdims=True)
    mask1 = lane == i1
    p2 = jnp.where(mask1, -1.0, p)
    m2 = jnp.max(p2, axis=1, keepdims=True)
    i2 = jnp.min(jnp.where(p2 >= m2, lane, e), axis=1, keepdims=True)
    mask2 = lane == i2
    denom = m1 + m2 + 1e-9
    w1_ref[...] = m1 / denom
    w2_ref[...] = m2 / denom

    mp = jnp.mean(p, axis=0)
    entropy = -jnp.sum(mp * jnp.log(mp + 1e-9))
    aux_ref[...] = jnp.full((1, 1), -entropy * AUX_COEF, jnp.float32)

    cnt = mask1.astype(jnp.float32) + mask2.astype(jnp.float32)
    cum = cnt
    row = jax.lax.broadcasted_iota(jnp.int32, (n, e), 0)
    shift = 1
    while shift < n:
        rolled = pltpu.roll(cum, shift, axis=0)
        cum = cum + jnp.where(row >= shift, rolled, 0.0)
        shift *= 2
    excl = cum - cnt
    counts = cum[-1:, :]
    ci = counts.astype(jnp.int32)
    padded = ((ci + TM - 1) // TM) * TM
    er = jax.lax.broadcasted_iota(jnp.int32, (e, e), 0)
    ec = jax.lax.broadcasted_iota(jnp.int32, (e, e), 1)
    tri = (ec < er).astype(jnp.int32)
    off = jnp.sum(tri * padded, axis=1)[None, :]
    offf = off.astype(jnp.float32)
    off1 = jnp.sum(jnp.where(mask1, offf, 0.0), axis=1, keepdims=True)
    off2 = jnp.sum(jnp.where(mask2, offf, 0.0), axis=1, keepdims=True)
    rank1 = jnp.sum(jnp.where(mask1, excl, 0.0), axis=1, keepdims=True)
    rank2 = jnp.sum(jnp.where(mask2, excl + mask1.astype(jnp.float32), 0.0),
                    axis=1, keepdims=True)
    s0_ref[...] = (off1 + rank1).astype(jnp.int32)
    s1_ref[...] = (off2 + rank2).astype(jnp.int32)

    tl = jax.lax.broadcasted_iota(jnp.int32, (1, N_TILES), 1)
    ends = off + padded
    cover = (ends[0:1, :].reshape(1, e, 1) <= (tl * TM).reshape(1, 1, N_TILES))
    ef = jnp.minimum(jnp.sum(cover.astype(jnp.int32), axis=1), e - 1)
    ef_ref[...] = ef
    ef_prev = pltpu.roll(ef, 1, axis=1)
    ff_ref[...] = ((tl == 0) | (ef != ef_prev)).astype(jnp.int32)


def _router(xf, router_w):
    n, d = xf.shape
    e = router_w.shape[0]
    return pl.pallas_call(
        _router_body,
        in_specs=[pl.BlockSpec((n, d), lambda: (0, 0)),
                  pl.BlockSpec((e, d), lambda: (0, 0))],
        out_specs=[pl.BlockSpec((1, 1), lambda: (0, 0)),
                   pl.BlockSpec((n, 1), lambda: (0, 0)),
                   pl.BlockSpec((n, 1), lambda: (0, 0)),
                   pl.BlockSpec((n, 1), lambda: (0, 0)),
                   pl.BlockSpec((n, 1), lambda: (0, 0)),
                   pl.BlockSpec((1, N_TILES), lambda: (0, 0)),
                   pl.BlockSpec((1, N_TILES), lambda: (0, 0))],
        out_shape=[jax.ShapeDtypeStruct((1, 1), jnp.float32),
                   jax.ShapeDtypeStruct((n, 1), jnp.int32),
                   jax.ShapeDtypeStruct((n, 1), jnp.int32),
                   jax.ShapeDtypeStruct((n, 1), jnp.float32),
                   jax.ShapeDtypeStruct((n, 1), jnp.float32),
                   jax.ShapeDtypeStruct((1, N_TILES), jnp.int32),
                   jax.ShapeDtypeStruct((1, N_TILES), jnp.int32)],
    )(xf, router_w)


def _gather_body(s0r_ref, s1r_ref, xb_ref, xs_ref):
    t = pl.program_id(0)
    rid = t * TM + jax.lax.broadcasted_iota(jnp.int32, (TM, 1), 0)
    s0 = s0r_ref[...]
    s1 = s1r_ref[...]
    sel = ((rid == s0).astype(jnp.bfloat16)
           + (rid == s1).astype(jnp.bfloat16))
    dn = (((1,), (0,)), ((), ()))
    xs_ref[...] = jax.lax.dot_general(
        sel, xb_ref[...], dn,
        preferred_element_type=jnp.float32).astype(jnp.bfloat16)


def _gather(s0r, s1r, xb):
    n, d = xb.shape
    return pl.pallas_call(
        _gather_body,
        grid=(N_TILES,),
        in_specs=[pl.BlockSpec((1, n), lambda t: (0, 0)),
                  pl.BlockSpec((1, n), lambda t: (0, 0)),
                  pl.BlockSpec((n, d), lambda t: (0, 0))],
        out_specs=pl.BlockSpec((TM, d), lambda t: (t, 0)),
        out_shape=jax.ShapeDtypeStruct((L_SLOTS, d), jnp.bfloat16),
        compiler_params=pltpu.CompilerParams(
            dimension_semantics=("arbitrary",)),
    )(s0r, s1r, xb)


def _expert_body(ef_ref, ff_ref, xs_ref, wg_ref, wu_ref, wd_ref,
                 gg_ref, ug_ref, dg_ref, ys_ref, wgq_s, wuq_s, wdq_s):
    t = pl.program_id(0)
    e = ef_ref[t]

    @pl.when(ff_ref[t] == 1)
    def _():
        for src, dst in ((wg_ref, wgq_s), (wu_ref, wuq_s), (wd_ref, wdq_s)):
            a, b = dst.shape
            nc = 8
            ch = a // nc
            s = jnp.float32(0.0)
            for i in range(nc):
                s += jnp.sum(jnp.abs(src[0, i * ch:(i + 1) * ch, :]))
            th = 0.5 * s / (a * b)
            for i in range(nc):
                w = src[0, i * ch:(i + 1) * ch, :]
                dst[i * ch:(i + 1) * ch, :] = (
                    (w > th).astype(jnp.float32)
                    - (w < -th).astype(jnp.float32)).astype(jnp.bfloat16)

    x = xs_ref[...]
    dn = (((1,), (1,)), ((), ()))
    g = jax.lax.dot_general(x, wgq_s[...], dn,
                            preferred_element_type=jnp.float32) * gg_ref[e]
    u = jax.lax.dot_general(x, wuq_s[...], dn,
                            preferred_element_type=jnp.float32) * ug_ref[e]
    h = (g * jax.nn.sigmoid(g) * u).astype(jnp.bfloat16)
    y = jax.lax.dot_general(h, wdq_s[...], dn,
                            preferred_element_type=jnp.float32) * dg_ref[e]
    ys_ref[...] = y.astype(jnp.bfloat16)


def _experts(ef, ff, xs, gate_w, up_w, down_w, gg, ug, dg):
    d = D_MODEL
    ff_dim = D_FF
    grid_spec = pltpu.PrefetchScalarGridSpec(
        num_scalar_prefetch=2,
        grid=(N_TILES,),
        in_specs=[
            pl.BlockSpec((TM, d), lambda t, ef_r, ff_r: (t, 0)),
            pl.BlockSpec((1, ff_dim, d), lambda t, ef_r, ff_r: (ef_r[t], 0, 0)),
            pl.BlockSpec((1, ff_dim, d), lambda t, ef_r, ff_r: (ef_r[t], 0, 0)),
            pl.BlockSpec((1, d, ff_dim), lambda t, ef_r, ff_r: (ef_r[t], 0, 0)),
            pl.BlockSpec(memory_space=pltpu.SMEM),
            pl.BlockSpec(memory_space=pltpu.SMEM),
            pl.BlockSpec(memory_space=pltpu.SMEM),
        ],
        out_specs=pl.BlockSpec((TM, d), lambda t, ef_r, ff_r: (t, 0)),
        scratch_shapes=[pltpu.VMEM((ff_dim, d), jnp.bfloat16),
                        pltpu.VMEM((ff_dim, d), jnp.bfloat16),
                        pltpu.VMEM((d, ff_dim), jnp.bfloat16)],
    )
    return pl.pallas_call(
        _expert_body,
        grid_spec=grid_spec,
        out_shape=jax.ShapeDtypeStruct((L_SLOTS, d), jnp.bfloat16),
        compiler_params=pltpu.CompilerParams(
            dimension_semantics=("arbitrary",),
            vmem_limit_bytes=100 * 1024 * 1024),
    )(ef, ff, xs, gate_w, up_w, down_w, gg, ug, dg)


def _combine_body(s0_ref, s1_ref, w1_ref, w2_ref, ys_ref, out_ref):
    s0 = s0_ref[...]
    s1 = s1_ref[...]
    w1 = w1_ref[...]
    w2 = w2_ref[...]
    sl = jax.lax.broadcasted_iota(jnp.int32, (1, L_SLOTS), 1)
    c = (jnp.where(s0 == sl, w1, 0.0)
         + jnp.where(s1 == sl, w2, 0.0)).astype(jnp.bfloat16)
    dn = (((1,), (0,)), ((), ()))
    out_ref[...] = jax.lax.dot_general(c, ys_ref[...], dn,
                                       preferred_element_type=jnp.float32)


def _combine(s0, s1, w1, w2, ys):
    n = N_TOKENS
    d = D_MODEL
    return pl.pallas_call(
        _combine_body,
        grid=(n // TM,),
        in_specs=[pl.BlockSpec((TM, 1), lambda m: (m, 0)),
                  pl.BlockSpec((TM, 1), lambda m: (m, 0)),
                  pl.BlockSpec((TM, 1), lambda m: (m, 0)),
                  pl.BlockSpec((TM, 1), lambda m: (m, 0)),
                  pl.BlockSpec((L_SLOTS, d), lambda m: (0, 0))],
        out_specs=pl.BlockSpec((TM, d), lambda m: (m, 0)),
        out_shape=jax.ShapeDtypeStruct((n, d), jnp.float32),
        compiler_params=pltpu.CompilerParams(
            dimension_semantics=("arbitrary",)),
    )(s0, s1, w1, w2, ys)


def kernel(x, router_w, gate_w, up_w, down_w, gate_gamma, up_gamma, down_gamma):
    b, t, d = x.shape
    xf = x.reshape(-1, d)
    aux, s0, s1, w1, w2, ef, ff = _router(xf, router_w)
    xb = xf.astype(jnp.bfloat16)
    xs = _gather(jnp.transpose(s0), jnp.transpose(s1), xb)
    ys = _experts(ef.reshape(-1), ff.reshape(-1), xs,
                  gate_w, up_w, down_w, gate_gamma, up_gamma, down_gamma)
    out = _combine(s0, s1, w1, w2, ys)
    return out.reshape(b, t, d), aux[0, 0]


# speedup vs baseline: 1.0860x; 1.0860x over previous
import jax
import jax.numpy as jnp
from jax.experimental import pallas as pl
from jax.experimental.pallas import tpu as pltpu

D_MODEL = 768
D_FF = 2048
N_EXPERTS = 8
TOP_K = 2
AUX_COEF = 0.01
N_TOKENS = 2048
TM = 128
L_SLOTS = N_TOKENS * TOP_K + N_EXPERTS * TM
N_TILES = L_SLOTS // TM


def _router_body(x_ref, rw_ref, aux_ref, s0_ref, s1_ref, w1_ref, w2_ref,
                 ef_ref, ff_ref, ne_ref, hn_ref, par_ref):
    x = x_ref[...]
    rw = rw_ref[...]
    dn = (((1,), (1,)), ((), ()))
    logits = jax.lax.dot_general(x, rw, dn, preferred_element_type=jnp.float32)
    p = jax.nn.softmax(logits, axis=-1)
    n, e = p.shape
    lane = jax.lax.broadcasted_iota(jnp.int32, (n, e), 1)
    m1 = jnp.max(p, axis=1, keepdims=True)
    i1 = jnp.min(jnp.where(p >= m1, lane, e), axis=1, keepdims=True)
    mask1 = lane == i1
    p2 = jnp.where(mask1, -1.0, p)
    m2 = jnp.max(p2, axis=1, keepdims=True)
    i2 = jnp.min(jnp.where(p2 >= m2, lane, e), axis=1, keepdims=True)
    mask2 = lane == i2
    denom = m1 + m2 + 1e-9
    w1_ref[...] = m1 / denom
    w2_ref[...] = m2 / denom

    mp = jnp.mean(p, axis=0)
    entropy = -jnp.sum(mp * jnp.log(mp + 1e-9))
    aux_ref[...] = jnp.full((1, 1), -entropy * AUX_COEF, jnp.float32)

    cnt = mask1.astype(jnp.float32) + mask2.astype(jnp.float32)
    cum = cnt
    row = jax.lax.broadcasted_iota(jnp.int32, (n, e), 0)
    shift = 1
    while shift < n:
        rolled = pltpu.roll(cum, shift, axis=0)
        cum = cum + jnp.where(row >= shift, rolled, 0.0)
        shift *= 2
    excl = cum - cnt
    counts = cum[-1:, :]
    ci = counts.astype(jnp.int32)
    padded = ((ci + TM - 1) // TM) * TM
    er = jax.lax.broadcasted_iota(jnp.int32, (e, e), 0)
    ec = jax.lax.broadcasted_iota(jnp.int32, (e, e), 1)
    tri = (ec < er).astype(jnp.int32)
    off = jnp.sum(tri * padded, axis=1)[None, :]
    offf = off.astype(jnp.float32)
    off1 = jnp.sum(jnp.where(mask1, offf, 0.0), axis=1, keepdims=True)
    off2 = jnp.sum(jnp.where(mask2, offf, 0.0), axis=1, keepdims=True)
    rank1 = jnp.sum(jnp.where(mask1, excl, 0.0), axis=1, keepdims=True)
    rank2 = jnp.sum(jnp.where(mask2, excl + mask1.astype(jnp.float32), 0.0),
                    axis=1, keepdims=True)
    s0_ref[...] = (off1 + rank1).astype(jnp.int32)
    s1_ref[...] = (off2 + rank2).astype(jnp.int32)

    tl = jax.lax.broadcasted_iota(jnp.int32, (1, N_TILES), 1)
    ends = off + padded
    cover = (ends[0:1, :].reshape(1, e, 1) <= (tl * TM).reshape(1, 1, N_TILES))
    ef = jnp.minimum(jnp.sum(cover.astype(jnp.int32), axis=1), e - 1)
    ef_ref[...] = ef
    ef_prev = pltpu.roll(ef, 1, axis=1)
    total = ends[0:1, e - 1:e]
    real = tl * TM < total
    ff = (((tl == 0) | (ef != ef_prev)) & real).astype(jnp.int32)
    ff_ref[...] = ff
    ends_sel = jnp.zeros((1, N_TILES), jnp.int32)
    for ei in range(e):
        ends_sel = jnp.where(ef == ei, ends[0:1, ei:ei + 1], ends_sel)
    ne = jnp.zeros((1, N_TILES), jnp.int32)
    for ei in range(e):
        ne = ne + (ends[0:1, ei:ei + 1] <= ends_sel).astype(jnp.int32)
    ne_ref[...] = jnp.minimum(ne, e - 1)
    hn_ref[...] = (ends_sel < total).astype(jnp.int32)
    c = ff
    shift = 1
    while shift < N_TILES:
        c = c + jnp.where(tl >= shift, pltpu.roll(c, shift, axis=1), 0)
        shift *= 2
    par_ref[...] = (c - 1) % 2


def _router(xf, router_w):
    n, d = xf.shape
    e = router_w.shape[0]
    return pl.pallas_call(
        _router_body,
        in_specs=[pl.BlockSpec((n, d), lambda: (0, 0)),
                  pl.BlockSpec((e, d), lambda: (0, 0))],
        out_specs=[pl.BlockSpec((1, 1), lambda: (0, 0)),
                   pl.BlockSpec((n, 1), lambda: (0, 0)),
                   pl.BlockSpec((n, 1), lambda: (0, 0)),
                   pl.BlockSpec((n, 1), lambda: (0, 0)),
                   pl.BlockSpec((n, 1), lambda: (0, 0)),
                   pl.BlockSpec((1, N_TILES), lambda: (0, 0)),
                   pl.BlockSpec((1, N_TILES), lambda: (0, 0)),
                   pl.BlockSpec((1, N_TILES), lambda: (0, 0)),
                   pl.BlockSpec((1, N_TILES), lambda: (0, 0)),
                   pl.BlockSpec((1, N_TILES), lambda: (0, 0))],
        out_shape=[jax.ShapeDtypeStruct((1, 1), jnp.float32),
                   jax.ShapeDtypeStruct((n, 1), jnp.int32),
                   jax.ShapeDtypeStruct((n, 1), jnp.int32),
                   jax.ShapeDtypeStruct((n, 1), jnp.float32),
                   jax.ShapeDtypeStruct((n, 1), jnp.float32),
                   jax.ShapeDtypeStruct((1, N_TILES), jnp.int32),
                   jax.ShapeDtypeStruct((1, N_TILES), jnp.int32),
                   jax.ShapeDtypeStruct((1, N_TILES), jnp.int32),
                   jax.ShapeDtypeStruct((1, N_TILES), jnp.int32),
                   jax.ShapeDtypeStruct((1, N_TILES), jnp.int32)],
    )(xf, router_w)


def _gather_body(s0r_ref, s1r_ref, xb_ref, xs_ref):
    t = pl.program_id(0)
    rid = t * TM + jax.lax.broadcasted_iota(jnp.int32, (TM, 1), 0)
    s0 = s0r_ref[...]
    s1 = s1r_ref[...]
    sel = ((rid == s0).astype(jnp.bfloat16)
           + (rid == s1).astype(jnp.bfloat16))
    dn = (((1,), (0,)), ((), ()))
    xs_ref[...] = jax.lax.dot_general(
        sel, xb_ref[...], dn,
        preferred_element_type=jnp.float32).astype(jnp.bfloat16)


def _gather(s0r, s1r, xb):
    n, d = xb.shape
    return pl.pallas_call(
        _gather_body,
        grid=(N_TILES,),
        in_specs=[pl.BlockSpec((1, n), lambda t: (0, 0)),
                  pl.BlockSpec((1, n), lambda t: (0, 0)),
                  pl.BlockSpec((n, d), lambda t: (0, 0))],
        out_specs=pl.BlockSpec((TM, d), lambda t: (t, 0)),
        out_shape=jax.ShapeDtypeStruct((L_SLOTS, d), jnp.bfloat16),
        compiler_params=pltpu.CompilerParams(
            dimension_semantics=("arbitrary",)),
    )(s0r, s1r, xb)


def _expert_body(ef_ref, ff_ref, ne_ref, hn_ref, par_ref,
                 xs_ref, wg_ref, wu_ref, wd_ref,
                 gg_ref, ug_ref, dg_ref, ys_ref,
                 rawg_s, rawu_s, rawd_s, wgq_s, wuq_s, wdq_s, sem):
    t = pl.program_id(0)
    e = ef_ref[t]

    def start_dma(ex, p):
        pltpu.make_async_copy(wg_ref.at[ex], rawg_s.at[p], sem).start()
        pltpu.make_async_copy(wu_ref.at[ex], rawu_s.at[p], sem).start()
        pltpu.make_async_copy(wd_ref.at[ex], rawd_s.at[p], sem).start()

    def wait_dma(ex, p):
        pltpu.make_async_copy(wg_ref.at[ex], rawg_s.at[p], sem).wait()
        pltpu.make_async_copy(wu_ref.at[ex], rawu_s.at[p], sem).wait()
        pltpu.make_async_copy(wd_ref.at[ex], rawd_s.at[p], sem).wait()

    @pl.when(ff_ref[t] == 1)
    def _():
        p = par_ref[t]

        @pl.when(t == 0)
        def _():
            start_dma(e, p)

        wait_dma(e, p)

        @pl.when(hn_ref[t] == 1)
        def _():
            start_dma(ne_ref[t], 1 - p)

        for src, dst in ((rawg_s, wgq_s), (rawu_s, wuq_s), (rawd_s, wdq_s)):
            a, b = dst.shape
            nc = 8
            ch = a // nc
            s = jnp.float32(0.0)
            for i in range(nc):
                s += jnp.sum(jnp.abs(src[p, i * ch:(i + 1) * ch, :]))
            th = 0.5 * s / (a * b)
            for i in range(nc):
                w = src[p, i * ch:(i + 1) * ch, :]
                dst[i * ch:(i + 1) * ch, :] = (
                    (w > th).astype(jnp.float32)
                    - (w < -th).astype(jnp.float32)).astype(jnp.bfloat16)

    x = xs_ref[...]
    dn = (((1,), (1,)), ((), ()))
    g = jax.lax.dot_general(x, wgq_s[...], dn,
                            preferred_element_type=jnp.float32) * gg_ref[e]
    u = jax.lax.dot_general(x, wuq_s[...], dn,
                            preferred_element_type=jnp.float32) * ug_ref[e]
    h = (g * jax.nn.sigmoid(g) * u).astype(jnp.bfloat16)
    y = jax.lax.dot_general(h, wdq_s[...], dn,
                            preferred_element_type=jnp.float32) * dg_ref[e]
    ys_ref[...] = y.astype(jnp.bfloat16)


def _experts(ef, ff, ne, hn, par, xs, gate_w, up_w, down_w, gg, ug, dg):
    d = D_MODEL
    ff_dim = D_FF
    grid_spec = pltpu.PrefetchScalarGridSpec(
        num_scalar_prefetch=5,
        grid=(N_TILES,),
        in_specs=[
            pl.BlockSpec((TM, d), lambda t, *_: (t, 0)),
            pl.BlockSpec(memory_space=pl.ANY),
            pl.BlockSpec(memory_space=pl.ANY),
            pl.BlockSpec(memory_space=pl.ANY),
            pl.BlockSpec(memory_space=pltpu.SMEM),
            pl.BlockSpec(memory_space=pltpu.SMEM),
            pl.BlockSpec(memory_space=pltpu.SMEM),
        ],
        out_specs=pl.BlockSpec((TM, d), lambda t, *_: (t, 0)),
        scratch_shapes=[pltpu.VMEM((2, ff_dim, d), jnp.float32),
                        pltpu.VMEM((2, ff_dim, d), jnp.float32),
                        pltpu.VMEM((2, d, ff_dim), jnp.float32),
                        pltpu.VMEM((ff_dim, d), jnp.bfloat16),
                        pltpu.VMEM((ff_dim, d), jnp.bfloat16),
                        pltpu.VMEM((d, ff_dim), jnp.bfloat16),
                        pltpu.SemaphoreType.DMA],
    )
    return pl.pallas_call(
        _expert_body,
        grid_spec=grid_spec,
        out_shape=jax.ShapeDtypeStruct((L_SLOTS, d), jnp.bfloat16),
        compiler_params=pltpu.CompilerParams(
            dimension_semantics=("arbitrary",),
            vmem_limit_bytes=100 * 1024 * 1024),
    )(ef, ff, ne, hn, par, xs, gate_w, up_w, down_w, gg, ug, dg)


def _combine_body(s0_ref, s1_ref, w1_ref, w2_ref, ys_ref, out_ref):
    s0 = s0_ref[...]
    s1 = s1_ref[...]
    w1 = w1_ref[...]
    w2 = w2_ref[...]
    sl = jax.lax.broadcasted_iota(jnp.int32, (1, L_SLOTS), 1)
    c = (jnp.where(s0 == sl, w1, 0.0)
         + jnp.where(s1 == sl, w2, 0.0)).astype(jnp.bfloat16)
    dn = (((1,), (0,)), ((), ()))
    out_ref[...] = jax.lax.dot_general(c, ys_ref[...], dn,
                                       preferred_element_type=jnp.float32)


def _combine(s0, s1, w1, w2, ys):
    n = N_TOKENS
    d = D_MODEL
    return pl.pallas_call(
        _combine_body,
        grid=(n // TM,),
        in_specs=[pl.BlockSpec((TM, 1), lambda m: (m, 0)),
                  pl.BlockSpec((TM, 1), lambda m: (m, 0)),
                  pl.BlockSpec((TM, 1), lambda m: (m, 0)),
                  pl.BlockSpec((TM, 1), lambda m: (m, 0)),
                  pl.BlockSpec((L_SLOTS, d), lambda m: (0, 0))],
        out_specs=pl.BlockSpec((TM, d), lambda m: (m, 0)),
        out_shape=jax.ShapeDtypeStruct((n, d), jnp.float32),
        compiler_params=pltpu.CompilerParams(
            dimension_semantics=("arbitrary",)),
    )(s0, s1, w1, w2, ys)


def kernel(x, router_w, gate_w, up_w, down_w, gate_gamma, up_gamma, down_gamma):
    b, t, d = x.shape
    xf = x.reshape(-1, d)
    aux, s0, s1, w1, w2, ef, ff, ne, hn, par = _router(xf, router_w)
    xb = xf.astype(jnp.bfloat16)
    xs = _gather(jnp.transpose(s0), jnp.transpose(s1), xb)
    ys = _experts(ef.reshape(-1), ff.reshape(-1), ne.reshape(-1),
                  hn.reshape(-1), par.reshape(-1), xs,
                  gate_w, up_w, down_w, gate_gamma, up_gamma, down_gamma)
    out = _combine(s0, s1, w1, w2, ys)
    return out.reshape(b, t, d), aux[0, 0]


# speedup vs baseline: 1.1424x; 1.0519x over previous
import jax
import jax.numpy as jnp
from jax.experimental import pallas as pl
from jax.experimental.pallas import tpu as pltpu

D_MODEL = 768
D_FF = 2048
N_EXPERTS = 8
TOP_K = 2
AUX_COEF = 0.01
N_TOKENS = 2048
TM = 128
L_SLOTS = N_TOKENS * TOP_K + N_EXPERTS * TM
N_TILES = L_SLOTS // TM


def _router_body(x_ref, rw_ref, aux_ref, s0_ref, s1_ref, w1_ref, w2_ref,
                 ef_ref, ff_ref, ne_ref, hn_ref, par_ref):
    x = x_ref[...]
    rw = rw_ref[...]
    dn = (((1,), (1,)), ((), ()))
    logits = jax.lax.dot_general(x, rw, dn, preferred_element_type=jnp.float32)
    p = jax.nn.softmax(logits, axis=-1)
    n, e = p.shape
    lane = jax.lax.broadcasted_iota(jnp.int32, (n, e), 1)
    m1 = jnp.max(p, axis=1, keepdims=True)
    i1 = jnp.min(jnp.where(p >= m1, lane, e), axis=1, keepdims=True)
    mask1 = lane == i1
    p2 = jnp.where(mask1, -1.0, p)
    m2 = jnp.max(p2, axis=1, keepdims=True)
    i2 = jnp.min(jnp.where(p2 >= m2, lane, e), axis=1, keepdims=True)
    mask2 = lane == i2
    denom = m1 + m2 + 1e-9
    w1_ref[...] = m1 / denom
    w2_ref[...] = m2 / denom

    mp = jnp.mean(p, axis=0)
    entropy = -jnp.sum(mp * jnp.log(mp + 1e-9))
    aux_ref[...] = jnp.full((1, 1), -entropy * AUX_COEF, jnp.float32)

    cnt = mask1.astype(jnp.float32) + mask2.astype(jnp.float32)
    cum = cnt
    row = jax.lax.broadcasted_iota(jnp.int32, (n, e), 0)
    shift = 1
    while shift < n:
        rolled = pltpu.roll(cum, shift, axis=0)
        cum = cum + jnp.where(row >= shift, rolled, 0.0)
        shift *= 2
    excl = cum - cnt
    counts = cum[-1:, :]
    ci = counts.astype(jnp.int32)
    padded = ((ci + TM - 1) // TM) * TM
    er = jax.lax.broadcasted_iota(jnp.int32, (e, e), 0)
    ec = jax.lax.broadcasted_iota(jnp.int32, (e, e), 1)
    tri = (ec < er).astype(jnp.int32)
    off = jnp.sum(tri * padded, axis=1)[None, :]
    offf = off.astype(jnp.float32)
    off1 = jnp.sum(jnp.where(mask1, offf, 0.0), axis=1, keepdims=True)
    off2 = jnp.sum(jnp.where(mask2, offf, 0.0), axis=1, keepdims=True)
    rank1 = jnp.sum(jnp.where(mask1, excl, 0.0), axis=1, keepdims=True)
    rank2 = jnp.sum(jnp.where(mask2, excl + mask1.astype(jnp.float32), 0.0),
                    axis=1, keepdims=True)
    s0_ref[...] = (off1 + rank1).astype(jnp.int32)
    s1_ref[...] = (off2 + rank2).astype(jnp.int32)

    tl = jax.lax.broadcasted_iota(jnp.int32, (1, N_TILES), 1)
    ends = off + padded
    cover = (ends[0:1, :].reshape(1, e, 1) <= (tl * TM).reshape(1, 1, N_TILES))
    ef = jnp.minimum(jnp.sum(cover.astype(jnp.int32), axis=1), e - 1)
    ef_ref[...] = ef
    ef_prev = pltpu.roll(ef, 1, axis=1)
    total = ends[0:1, e - 1:e]
    real = tl * TM < total
    ff = (((tl == 0) | (ef != ef_prev)) & real).astype(jnp.int32)
    ff_ref[...] = ff
    ends_sel = jnp.zeros((1, N_TILES), jnp.int32)
    for ei in range(e):
        ends_sel = jnp.where(ef == ei, ends[0:1, ei:ei + 1], ends_sel)
    ne = jnp.zeros((1, N_TILES), jnp.int32)
    for ei in range(e):
        ne = ne + (ends[0:1, ei:ei + 1] <= ends_sel).astype(jnp.int32)
    ne_ref[...] = jnp.minimum(ne, e - 1)
    hn_ref[...] = (ends_sel < total).astype(jnp.int32)
    c = ff
    shift = 1
    while shift < N_TILES:
        c = c + jnp.where(tl >= shift, pltpu.roll(c, shift, axis=1), 0)
        shift *= 2
    par_ref[...] = (c - 1) % 2


def _router(xf, router_w):
    n, d = xf.shape
    e = router_w.shape[0]
    return pl.pallas_call(
        _router_body,
        in_specs=[pl.BlockSpec((n, d), lambda: (0, 0)),
                  pl.BlockSpec((e, d), lambda: (0, 0))],
        out_specs=[pl.BlockSpec((1, 1), lambda: (0, 0)),
                   pl.BlockSpec((n, 1), lambda: (0, 0)),
                   pl.BlockSpec((n, 1), lambda: (0, 0)),
                   pl.BlockSpec((n, 1), lambda: (0, 0)),
                   pl.BlockSpec((n, 1), lambda: (0, 0)),
                   pl.BlockSpec((1, N_TILES), lambda: (0, 0)),
                   pl.BlockSpec((1, N_TILES), lambda: (0, 0)),
                   pl.BlockSpec((1, N_TILES), lambda: (0, 0)),
                   pl.BlockSpec((1, N_TILES), lambda: (0, 0)),
                   pl.BlockSpec((1, N_TILES), lambda: (0, 0))],
        out_shape=[jax.ShapeDtypeStruct((1, 1), jnp.float32),
                   jax.ShapeDtypeStruct((n, 1), jnp.int32),
                   jax.ShapeDtypeStruct((n, 1), jnp.int32),
                   jax.ShapeDtypeStruct((n, 1), jnp.float32),
                   jax.ShapeDtypeStruct((n, 1), jnp.float32),
                   jax.ShapeDtypeStruct((1, N_TILES), jnp.int32),
                   jax.ShapeDtypeStruct((1, N_TILES), jnp.int32),
                   jax.ShapeDtypeStruct((1, N_TILES), jnp.int32),
                   jax.ShapeDtypeStruct((1, N_TILES), jnp.int32),
                   jax.ShapeDtypeStruct((1, N_TILES), jnp.int32)],
    )(xf, router_w)


def _gather_body(s0r_ref, s1r_ref, xb_ref, xs_ref):
    t = pl.program_id(0)
    rid = t * TM + jax.lax.broadcasted_iota(jnp.int32, (TM, 1), 0)
    s0 = s0r_ref[...]
    s1 = s1r_ref[...]
    sel = ((rid == s0).astype(jnp.bfloat16)
           + (rid == s1).astype(jnp.bfloat16))
    dn = (((1,), (0,)), ((), ()))
    xs_ref[...] = jax.lax.dot_general(
        sel, xb_ref[...], dn,
        preferred_element_type=jnp.float32).astype(jnp.bfloat16)


def _gather(s0r, s1r, xb):
    n, d = xb.shape
    return pl.pallas_call(
        _gather_body,
        grid=(N_TILES,),
        in_specs=[pl.BlockSpec((1, n), lambda t: (0, 0)),
                  pl.BlockSpec((1, n), lambda t: (0, 0)),
                  pl.BlockSpec((n, d), lambda t: (0, 0))],
        out_specs=pl.BlockSpec((TM, d), lambda t: (t, 0)),
        out_shape=jax.ShapeDtypeStruct((L_SLOTS, d), jnp.bfloat16),
        compiler_params=pltpu.CompilerParams(
            dimension_semantics=("arbitrary",)),
    )(s0r, s1r, xb)


def _expert_body(ef_ref, ff_ref, ne_ref, hn_ref, par_ref,
                 s0r_ref, s1r_ref, xb_ref, wg_ref, wu_ref, wd_ref,
                 gg_ref, ug_ref, dg_ref, ys_ref,
                 rawg_s, rawu_s, rawd_s, wgq_s, wuq_s, wdq_s, sem):
    t = pl.program_id(0)
    e = ef_ref[t]

    def start_dma(ex, p):
        pltpu.make_async_copy(wg_ref.at[ex], rawg_s.at[p], sem).start()
        pltpu.make_async_copy(wu_ref.at[ex], rawu_s.at[p], sem).start()
        pltpu.make_async_copy(wd_ref.at[ex], rawd_s.at[p], sem).start()

    def wait_dma(ex, p):
        pltpu.make_async_copy(wg_ref.at[ex], rawg_s.at[p], sem).wait()
        pltpu.make_async_copy(wu_ref.at[ex], rawu_s.at[p], sem).wait()
        pltpu.make_async_copy(wd_ref.at[ex], rawd_s.at[p], sem).wait()

    @pl.when(ff_ref[t] == 1)
    def _():
        p = par_ref[t]

        @pl.when(t == 0)
        def _():
            start_dma(e, p)

        wait_dma(e, p)

        @pl.when(hn_ref[t] == 1)
        def _():
            start_dma(ne_ref[t], 1 - p)

        for src, dst in ((rawg_s, wgq_s), (rawu_s, wuq_s), (rawd_s, wdq_s)):
            a, b = dst.shape
            nc = 8
            ch = a // nc
            s = jnp.float32(0.0)
            for i in range(nc):
                s += jnp.sum(jnp.abs(src[p, i * ch:(i + 1) * ch, :]))
            th = 0.5 * s / (a * b)
            for i in range(nc):
                w = src[p, i * ch:(i + 1) * ch, :]
                dst[i * ch:(i + 1) * ch, :] = (
                    (w > th).astype(jnp.float32)
                    - (w < -th).astype(jnp.float32)).astype(jnp.bfloat16)

    rid = t * TM + jax.lax.broadcasted_iota(jnp.int32, (TM, 1), 0)
    sel = ((rid == s0r_ref[...]).astype(jnp.bfloat16)
           + (rid == s1r_ref[...]).astype(jnp.bfloat16))
    dng = (((1,), (0,)), ((), ()))
    x = jax.lax.dot_general(
        sel, xb_ref[...], dng,
        preferred_element_type=jnp.float32).astype(jnp.bfloat16)
    dn = (((1,), (1,)), ((), ()))
    g = jax.lax.dot_general(x, wgq_s[...], dn,
                            preferred_element_type=jnp.float32) * gg_ref[e]
    u = jax.lax.dot_general(x, wuq_s[...], dn,
                            preferred_element_type=jnp.float32) * ug_ref[e]
    h = (g * jax.nn.sigmoid(g) * u).astype(jnp.bfloat16)
    y = jax.lax.dot_general(h, wdq_s[...], dn,
                            preferred_element_type=jnp.float32) * dg_ref[e]
    ys_ref[...] = y.astype(jnp.bfloat16)


def _experts(ef, ff, ne, hn, par, s0r, s1r, xb,
             gate_w, up_w, down_w, gg, ug, dg):
    d = D_MODEL
    n = N_TOKENS
    ff_dim = D_FF
    grid_spec = pltpu.PrefetchScalarGridSpec(
        num_scalar_prefetch=5,
        grid=(N_TILES,),
        in_specs=[
            pl.BlockSpec((1, n), lambda t, *_: (0, 0)),
            pl.BlockSpec((1, n), lambda t, *_: (0, 0)),
            pl.BlockSpec((n, d), lambda t, *_: (0, 0)),
            pl.BlockSpec(memory_space=pl.ANY),
            pl.BlockSpec(memory_space=pl.ANY),
            pl.BlockSpec(memory_space=pl.ANY),
            pl.BlockSpec(memory_space=pltpu.SMEM),
            pl.BlockSpec(memory_space=pltpu.SMEM),
            pl.BlockSpec(memory_space=pltpu.SMEM),
        ],
        out_specs=pl.BlockSpec((TM, d), lambda t, *_: (t, 0)),
        scratch_shapes=[pltpu.VMEM((2, ff_dim, d), jnp.float32),
                        pltpu.VMEM((2, ff_dim, d), jnp.float32),
                        pltpu.VMEM((2, d, ff_dim), jnp.float32),
                        pltpu.VMEM((ff_dim, d), jnp.bfloat16),
                        pltpu.VMEM((ff_dim, d), jnp.bfloat16),
                        pltpu.VMEM((d, ff_dim), jnp.bfloat16),
                        pltpu.SemaphoreType.DMA],
    )
    return pl.pallas_call(
        _expert_body,
        grid_spec=grid_spec,
        out_shape=jax.ShapeDtypeStruct((L_SLOTS, d), jnp.bfloat16),
        compiler_params=pltpu.CompilerParams(
            dimension_semantics=("arbitrary",),
            vmem_limit_bytes=100 * 1024 * 1024),
    )(ef, ff, ne, hn, par, s0r, s1r, xb, gate_w, up_w, down_w, gg, ug, dg)


def _combine_body(s0_ref, s1_ref, w1_ref, w2_ref, ys_ref, out_ref):
    s0 = s0_ref[...]
    s1 = s1_ref[...]
    w1 = w1_ref[...]
    w2 = w2_ref[...]
    sl = jax.lax.broadcasted_iota(jnp.int32, (1, L_SLOTS), 1)
    c = (jnp.where(s0 == sl, w1, 0.0)
         + jnp.where(s1 == sl, w2, 0.0)).astype(jnp.bfloat16)
    dn = (((1,), (0,)), ((), ()))
    out_ref[...] = jax.lax.dot_general(c, ys_ref[...], dn,
                                       preferred_element_type=jnp.float32)


def _combine(s0, s1, w1, w2, ys):
    n = N_TOKENS
    d = D_MODEL
    return pl.pallas_call(
        _combine_body,
        grid=(n // TM,),
        in_specs=[pl.BlockSpec((TM, 1), lambda m: (m, 0)),
                  pl.BlockSpec((TM, 1), lambda m: (m, 0)),
                  pl.BlockSpec((TM, 1), lambda m: (m, 0)),
                  pl.BlockSpec((TM, 1), lambda m: (m, 0)),
                  pl.BlockSpec((L_SLOTS, d), lambda m: (0, 0))],
        out_specs=pl.BlockSpec((TM, d), lambda m: (m, 0)),
        out_shape=jax.ShapeDtypeStruct((n, d), jnp.float32),
        compiler_params=pltpu.CompilerParams(
            dimension_semantics=("arbitrary",)),
    )(s0, s1, w1, w2, ys)


def kernel(x, router_w, gate_w, up_w, down_w, gate_gamma, up_gamma, down_gamma):
    b, t, d = x.shape
    xf = x.reshape(-1, d)
    aux, s0, s1, w1, w2, ef, ff, ne, hn, par = _router(xf, router_w)
    xb = xf.astype(jnp.bfloat16)
    ys = _experts(ef.reshape(-1), ff.reshape(-1), ne.reshape(-1),
                  hn.reshape(-1), par.reshape(-1),
                  jnp.transpose(s0), jnp.transpose(s1), xb,
                  gate_w, up_w, down_w, gate_gamma, up_gamma, down_gamma)
    out = _combine(s0, s1, w1, w2, ys)
    return out.reshape(b, t, d), aux[0, 0]


# speedup vs baseline: 1.1522x; 1.0086x over previous
import jax
import jax.numpy as jnp
from jax.experimental import pallas as pl
from jax.experimental.pallas import tpu as pltpu

D_MODEL = 768
D_FF = 2048
N_EXPERTS = 8
TOP_K = 2
AUX_COEF = 0.01
N_TOKENS = 2048
TM = 128
L_SLOTS = N_TOKENS * TOP_K + N_EXPERTS * TM
N_TILES = L_SLOTS // TM


def _router_body(x_ref, rw_ref, aux_ref, s0_ref, s1_ref, w1_ref, w2_ref,
                 ef_ref, ff_ref, ne_ref, hn_ref, par_ref):
    x = x_ref[...]
    rw = rw_ref[...]
    dn = (((1,), (1,)), ((), ()))
    logits = jax.lax.dot_general(x, rw, dn, preferred_element_type=jnp.float32)
    p = jax.nn.softmax(logits, axis=-1)
    n, e = p.shape
    lane = jax.lax.broadcasted_iota(jnp.int32, (n, e), 1)
    m1 = jnp.max(p, axis=1, keepdims=True)
    i1 = jnp.min(jnp.where(p >= m1, lane, e), axis=1, keepdims=True)
    mask1 = lane == i1
    p2 = jnp.where(mask1, -1.0, p)
    m2 = jnp.max(p2, axis=1, keepdims=True)
    i2 = jnp.min(jnp.where(p2 >= m2, lane, e), axis=1, keepdims=True)
    mask2 = lane == i2
    denom = m1 + m2 + 1e-9
    w1_ref[...] = m1 / denom
    w2_ref[...] = m2 / denom

    mp = jnp.mean(p, axis=0)
    entropy = -jnp.sum(mp * jnp.log(mp + 1e-9))
    aux_ref[...] = jnp.full((1, 1), -entropy * AUX_COEF, jnp.float32)

    cnt = mask1.astype(jnp.float32) + mask2.astype(jnp.float32)
    cum = cnt
    row = jax.lax.broadcasted_iota(jnp.int32, (n, e), 0)
    shift = 1
    while shift < n:
        rolled = pltpu.roll(cum, shift, axis=0)
        cum = cum + jnp.where(row >= shift, rolled, 0.0)
        shift *= 2
    excl = cum - cnt
    counts = cum[-1:, :]
    ci = counts.astype(jnp.int32)
    padded = ((ci + TM - 1) // TM) * TM
    er = jax.lax.broadcasted_iota(jnp.int32, (e, e), 0)
    ec = jax.lax.broadcasted_iota(jnp.int32, (e, e), 1)
    tri = (ec < er).astype(jnp.int32)
    off = jnp.sum(tri * padded, axis=1)[None, :]
    offf = off.astype(jnp.float32)
    off1 = jnp.sum(jnp.where(mask1, offf, 0.0), axis=1, keepdims=True)
    off2 = jnp.sum(jnp.where(mask2, offf, 0.0), axis=1, keepdims=True)
    rank1 = jnp.sum(jnp.where(mask1, excl, 0.0), axis=1, keepdims=True)
    rank2 = jnp.sum(jnp.where(mask2, excl + mask1.astype(jnp.float32), 0.0),
                    axis=1, keepdims=True)
    s0_ref[...] = (off1 + rank1).astype(jnp.int32)
    s1_ref[...] = (off2 + rank2).astype(jnp.int32)

    tl = jax.lax.broadcasted_iota(jnp.int32, (1, N_TILES), 1)
    ends = off + padded
    cover = (ends[0:1, :].reshape(1, e, 1) <= (tl * TM).reshape(1, 1, N_TILES))
    ef = jnp.minimum(jnp.sum(cover.astype(jnp.int32), axis=1), e - 1)
    ef_ref[...] = ef
    ef_prev = pltpu.roll(ef, 1, axis=1)
    total = ends[0:1, e - 1:e]
    real = tl * TM < total
    ff = (((tl == 0) | (ef != ef_prev)) & real).astype(jnp.int32)
    ff_ref[...] = ff
    ends_sel = jnp.zeros((1, N_TILES), jnp.int32)
    for ei in range(e):
        ends_sel = jnp.where(ef == ei, ends[0:1, ei:ei + 1], ends_sel)
    ne = jnp.zeros((1, N_TILES), jnp.int32)
    for ei in range(e):
        ne = ne + (ends[0:1, ei:ei + 1] <= ends_sel).astype(jnp.int32)
    ne_ref[...] = jnp.minimum(ne, e - 1)
    hn_ref[...] = (ends_sel < total).astype(jnp.int32)
    c = ff
    shift = 1
    while shift < N_TILES:
        c = c + jnp.where(tl >= shift, pltpu.roll(c, shift, axis=1), 0)
        shift *= 2
    par_ref[...] = (c - 1) % 2


def _router(xf, router_w):
    n, d = xf.shape
    e = router_w.shape[0]
    return pl.pallas_call(
        _router_body,
        in_specs=[pl.BlockSpec((n, d), lambda: (0, 0)),
                  pl.BlockSpec((e, d), lambda: (0, 0))],
        out_specs=[pl.BlockSpec((1, 1), lambda: (0, 0)),
                   pl.BlockSpec((n, 1), lambda: (0, 0)),
                   pl.BlockSpec((n, 1), lambda: (0, 0)),
                   pl.BlockSpec((n, 1), lambda: (0, 0)),
                   pl.BlockSpec((n, 1), lambda: (0, 0)),
                   pl.BlockSpec((1, N_TILES), lambda: (0, 0)),
                   pl.BlockSpec((1, N_TILES), lambda: (0, 0)),
                   pl.BlockSpec((1, N_TILES), lambda: (0, 0)),
                   pl.BlockSpec((1, N_TILES), lambda: (0, 0)),
                   pl.BlockSpec((1, N_TILES), lambda: (0, 0))],
        out_shape=[jax.ShapeDtypeStruct((1, 1), jnp.float32),
                   jax.ShapeDtypeStruct((n, 1), jnp.int32),
                   jax.ShapeDtypeStruct((n, 1), jnp.int32),
                   jax.ShapeDtypeStruct((n, 1), jnp.float32),
                   jax.ShapeDtypeStruct((n, 1), jnp.float32),
                   jax.ShapeDtypeStruct((1, N_TILES), jnp.int32),
                   jax.ShapeDtypeStruct((1, N_TILES), jnp.int32),
                   jax.ShapeDtypeStruct((1, N_TILES), jnp.int32),
                   jax.ShapeDtypeStruct((1, N_TILES), jnp.int32),
                   jax.ShapeDtypeStruct((1, N_TILES), jnp.int32)],
    )(xf, router_w)


def _gather_body(s0r_ref, s1r_ref, xb_ref, xs_ref):
    t = pl.program_id(0)
    rid = t * TM + jax.lax.broadcasted_iota(jnp.int32, (TM, 1), 0)
    s0 = s0r_ref[...]
    s1 = s1r_ref[...]
    sel = ((rid == s0).astype(jnp.bfloat16)
           + (rid == s1).astype(jnp.bfloat16))
    dn = (((1,), (0,)), ((), ()))
    xs_ref[...] = jax.lax.dot_general(
        sel, xb_ref[...], dn,
        preferred_element_type=jnp.float32).astype(jnp.bfloat16)


def _gather(s0r, s1r, xb):
    n, d = xb.shape
    return pl.pallas_call(
        _gather_body,
        grid=(N_TILES,),
        in_specs=[pl.BlockSpec((1, n), lambda t: (0, 0)),
                  pl.BlockSpec((1, n), lambda t: (0, 0)),
                  pl.BlockSpec((n, d), lambda t: (0, 0))],
        out_specs=pl.BlockSpec((TM, d), lambda t: (t, 0)),
        out_shape=jax.ShapeDtypeStruct((L_SLOTS, d), jnp.bfloat16),
        compiler_params=pltpu.CompilerParams(
            dimension_semantics=("arbitrary",)),
    )(s0r, s1r, xb)


def _expert_body(ef_ref, ff_ref, ne_ref, hn_ref, par_ref,
                 s0r_ref, s1r_ref, xb_ref, wg_ref, wu_ref, wd_ref,
                 gg_ref, ug_ref, dg_ref, ys_ref,
                 rawg_s, rawu_s, rawd_s, wgq_s, wuq_s, wdq_s, sem):
    t = pl.program_id(0)
    e = ef_ref[t]

    def start_dma(ex, p):
        pltpu.make_async_copy(wg_ref.at[ex], rawg_s.at[p], sem).start()
        pltpu.make_async_copy(wu_ref.at[ex], rawu_s.at[p], sem).start()
        pltpu.make_async_copy(wd_ref.at[ex], rawd_s.at[p], sem).start()

    def wait_dma(ex, p):
        pltpu.make_async_copy(wg_ref.at[ex], rawg_s.at[p], sem).wait()
        pltpu.make_async_copy(wu_ref.at[ex], rawu_s.at[p], sem).wait()
        pltpu.make_async_copy(wd_ref.at[ex], rawd_s.at[p], sem).wait()

    @pl.when(ff_ref[t] == 1)
    def _():
        p = par_ref[t]

        @pl.when(t == 0)
        def _():
            start_dma(e, p)

        wait_dma(e, p)

        @pl.when(hn_ref[t] == 1)
        def _():
            start_dma(ne_ref[t], 1 - p)

        one = jnp.bfloat16(1)
        zero = jnp.bfloat16(0)
        neg1 = jnp.bfloat16(-1)
        for src, dst in ((rawg_s, wgq_s), (rawu_s, wuq_s), (rawd_s, wdq_s)):
            a, b = dst.shape
            nc = 8
            ch = a // nc
            ones = jnp.full((8, ch), jnp.bfloat16(1))
            dnm = (((1,), (0,)), ((), ()))
            s = jnp.zeros((8, b), jnp.float32)
            for i in range(nc):
                ab = jnp.abs(src[p, i * ch:(i + 1) * ch, :]).astype(jnp.bfloat16)
                s += jax.lax.dot_general(ones, ab, dnm,
                                         preferred_element_type=jnp.float32)
            th = 0.5 * jnp.sum(s[0:1, :]) / (a * b)
            for i in range(nc):
                w = src[p, i * ch:(i + 1) * ch, :]
                dst[i * ch:(i + 1) * ch, :] = jnp.where(
                    w > th, 1.0, jnp.where(w < -th, -1.0, 0.0)
                ).astype(jnp.bfloat16)

    rid = t * TM + jax.lax.broadcasted_iota(jnp.int32, (TM, 1), 0)
    sel = ((rid == s0r_ref[...]).astype(jnp.bfloat16)
           + (rid == s1r_ref[...]).astype(jnp.bfloat16))
    dng = (((1,), (0,)), ((), ()))
    x = jax.lax.dot_general(
        sel, xb_ref[...], dng,
        preferred_element_type=jnp.float32).astype(jnp.bfloat16)
    dn = (((1,), (1,)), ((), ()))
    g = jax.lax.dot_general(x, wgq_s[...], dn,
                            preferred_element_type=jnp.float32) * gg_ref[e]
    u = jax.lax.dot_general(x, wuq_s[...], dn,
                            preferred_element_type=jnp.float32) * ug_ref[e]
    h = (g * jax.nn.sigmoid(g) * u).astype(jnp.bfloat16)
    y = jax.lax.dot_general(h, wdq_s[...], dn,
                            preferred_element_type=jnp.float32) * dg_ref[e]
    ys_ref[...] = y.astype(jnp.bfloat16)


def _experts(ef, ff, ne, hn, par, s0r, s1r, xb,
             gate_w, up_w, down_w, gg, ug, dg):
    d = D_MODEL
    n = N_TOKENS
    ff_dim = D_FF
    grid_spec = pltpu.PrefetchScalarGridSpec(
        num_scalar_prefetch=5,
        grid=(N_TILES,),
        in_specs=[
            pl.BlockSpec((1, n), lambda t, *_: (0, 0)),
            pl.BlockSpec((1, n), lambda t, *_: (0, 0)),
            pl.BlockSpec((n, d), lambda t, *_: (0, 0)),
            pl.BlockSpec(memory_space=pl.ANY),
            pl.BlockSpec(memory_space=pl.ANY),
            pl.BlockSpec(memory_space=pl.ANY),
            pl.BlockSpec(memory_space=pltpu.SMEM),
            pl.BlockSpec(memory_space=pltpu.SMEM),
            pl.BlockSpec(memory_space=pltpu.SMEM),
        ],
        out_specs=pl.BlockSpec((TM, d), lambda t, *_: (t, 0)),
        scratch_shapes=[pltpu.VMEM((2, ff_dim, d), jnp.float32),
                        pltpu.VMEM((2, ff_dim, d), jnp.float32),
                        pltpu.VMEM((2, d, ff_dim), jnp.float32),
                        pltpu.VMEM((ff_dim, d), jnp.bfloat16),
                        pltpu.VMEM((ff_dim, d), jnp.bfloat16),
                        pltpu.VMEM((d, ff_dim), jnp.bfloat16),
                        pltpu.SemaphoreType.DMA],
    )
    return pl.pallas_call(
        _expert_body,
        grid_spec=grid_spec,
        out_shape=jax.ShapeDtypeStruct((L_SLOTS, d), jnp.bfloat16),
        compiler_params=pltpu.CompilerParams(
            dimension_semantics=("arbitrary",),
            vmem_limit_bytes=100 * 1024 * 1024),
    )(ef, ff, ne, hn, par, s0r, s1r, xb, gate_w, up_w, down_w, gg, ug, dg)


def _combine_body(s0_ref, s1_ref, w1_ref, w2_ref, ys_ref, out_ref):
    s0 = s0_ref[...]
    s1 = s1_ref[...]
    w1 = w1_ref[...]
    w2 = w2_ref[...]
    sl = jax.lax.broadcasted_iota(jnp.int32, (1, L_SLOTS), 1)
    c = (jnp.where(s0 == sl, w1, 0.0)
         + jnp.where(s1 == sl, w2, 0.0)).astype(jnp.bfloat16)
    dn = (((1,), (0,)), ((), ()))
    out_ref[...] = jax.lax.dot_general(c, ys_ref[...], dn,
                                       preferred_element_type=jnp.float32)


def _combine(s0, s1, w1, w2, ys):
    n = N_TOKENS
    d = D_MODEL
    return pl.pallas_call(
        _combine_body,
        grid=(n // TM,),
        in_specs=[pl.BlockSpec((TM, 1), lambda m: (m, 0)),
                  pl.BlockSpec((TM, 1), lambda m: (m, 0)),
                  pl.BlockSpec((TM, 1), lambda m: (m, 0)),
                  pl.BlockSpec((TM, 1), lambda m: (m, 0)),
                  pl.BlockSpec((L_SLOTS, d), lambda m: (0, 0))],
        out_specs=pl.BlockSpec((TM, d), lambda m: (m, 0)),
        out_shape=jax.ShapeDtypeStruct((n, d), jnp.float32),
        compiler_params=pltpu.CompilerParams(
            dimension_semantics=("arbitrary",)),
    )(s0, s1, w1, w2, ys)


def kernel(x, router_w, gate_w, up_w, down_w, gate_gamma, up_gamma, down_gamma):
    b, t, d = x.shape
    xf = x.reshape(-1, d)
    aux, s0, s1, w1, w2, ef, ff, ne, hn, par = _router(xf, router_w)
    xb = xf.astype(jnp.bfloat16)
    ys = _experts(ef.reshape(-1), ff.reshape(-1), ne.reshape(-1),
                  hn.reshape(-1), par.reshape(-1),
                  jnp.transpose(s0), jnp.transpose(s1), xb,
                  gate_w, up_w, down_w, gate_gamma, up_gamma, down_gamma)
    out = _combine(s0, s1, w1, w2, ys)
    return out.reshape(b, t, d), aux[0, 0]
